# baseline (speedup 1.0000x reference)
import functools
import jax
import jax.numpy as jnp
from jax import lax
from jax.experimental import pallas as pl
from jax.experimental.pallas import tpu as pltpu
from jax.experimental.pallas import tpu_sc as plsc

BATCH = 16384
VOCAB = 1000
D_USER = 32
U_STRIDE = 1008
D_DOW = 7
D_HOD = 4
D_OUT = D_USER + D_DOW + D_HOD

H_BASE = 0
H_WORDS = D_HOD * 24
U_BASE = H_WORDS
N_CHUNK = 4
CHUNK_ROWS = D_USER // N_CHUNK
CHUNK_WORDS = CHUNK_ROWS * U_STRIDE
TAB_WORDS = H_WORDS + D_USER * U_STRIDE

NC = 2
NS = 16
NW = NC * NS
L = 16
B_PER_W = BATCH // NW
GROUPS = B_PER_W // L


def _sc_kernel(user_id_hbm, dow_hbm, hod_hbm, tab_hbm, out_hbm,
               uid_v, dow_v, hod_v, utab_v, htab_v, outbuf_v,
               s0, s1, s2, s3):
    wid = lax.axis_index("s") * NC + lax.axis_index("c")
    base = wid * B_PER_W

    sems = [s0, s1, s2, s3]
    cps = [
        pltpu.async_copy(
            tab_hbm.at[pl.ds(U_BASE + k * CHUNK_WORDS, CHUNK_WORDS)],
            utab_v.at[pl.ds(k * CHUNK_WORDS, CHUNK_WORDS)],
            sems[k])
        for k in range(N_CHUNK)
    ]

    pltpu.sync_copy(user_id_hbm.at[pl.ds(base, B_PER_W)], uid_v)
    pltpu.sync_copy(dow_hbm.at[pl.ds(base, B_PER_W)], dow_v)
    pltpu.sync_copy(hod_hbm.at[pl.ds(base, B_PER_W)], hod_v)
    pltpu.sync_copy(tab_hbm.at[pl.ds(H_BASE, H_WORDS)], htab_v)

    lane = lax.iota(jnp.int32, L)
    zeros = jnp.zeros((L,), jnp.float32)
    ones = jnp.ones((L,), jnp.float32)

    @plsc.parallel_loop(0, GROUPS, unroll=4)
    def _prep(g):
        for c in range(D_DOW):
            outbuf_v[D_USER + c, pl.ds(g * L, L)] = zeros
        cols = g * L + lane
        d = dow_v[pl.ds(g * L, L)]
        plsc.store_scatter(outbuf_v, [D_USER + d, cols], ones)
        hsrc = jnp.minimum(hod_v[pl.ds(g * L, L)] + 1, 23)
        for j in range(D_HOD):
            vals = plsc.load_gather(htab_v, [j * 24 + hsrc])
            outbuf_v[D_USER + D_DOW + j, pl.ds(g * L, L)] = vals

    for k in range(N_CHUNK):
        cps[k].wait()

        @plsc.parallel_loop(0, GROUPS, unroll=4)
        def _user(g, _k=k):
            uidx = uid_v[pl.ds(g * L, L)] + 1
            for j in range(_k * CHUNK_ROWS, (_k + 1) * CHUNK_ROWS):
                vals = plsc.load_gather(utab_v, [j * U_STRIDE + uidx])
                outbuf_v[j, pl.ds(g * L, L)] = vals

    pltpu.sync_copy(outbuf_v, out_hbm.at[:, pl.ds(base, B_PER_W)])


@jax.jit
def kernel(user_id, dow, hod, user_table, hod_table):
    mesh = plsc.VectorSubcoreMesh(core_axis_name="c", subcore_axis_name="s")
    run = functools.partial(
        pl.kernel, mesh=mesh,
        compiler_params=pltpu.CompilerParams(needs_layout_passes=False),
        out_type=jax.ShapeDtypeStruct((D_OUT, BATCH), jnp.float32),
        scratch_types=[
            pltpu.VMEM((B_PER_W,), jnp.int32),
            pltpu.VMEM((B_PER_W,), jnp.int32),
            pltpu.VMEM((B_PER_W,), jnp.int32),
            pltpu.VMEM((D_USER * U_STRIDE,), jnp.float32),
            pltpu.VMEM((H_WORDS,), jnp.float32),
            pltpu.VMEM((D_OUT, B_PER_W), jnp.float32),
            pltpu.SemaphoreType.DMA,
            pltpu.SemaphoreType.DMA,
            pltpu.SemaphoreType.DMA,
            pltpu.SemaphoreType.DMA,
        ],
    )(_sc_kernel)
    tab = jnp.concatenate([
        hod_table.T.reshape(-1),
        jnp.pad(user_table.T, ((0, 0), (0, U_STRIDE - (VOCAB + 1)))).reshape(-1),
    ])
    out_t = run(user_id, dow, hod, tab)
    return out_t.T
